# baseline (speedup 1.0000x reference)
import functools

import jax
import jax.numpy as jnp
from jax import lax
from jax.experimental import pallas as pl
from jax.experimental.pallas import tpu as pltpu
from jax.experimental.pallas import tpu_sc as plsc

_L = 16


@functools.cache
def _make_kernel(b, h, v, d):
    info = plsc.get_sparse_core_info()
    nc, ns = info.num_cores, info.num_subcores
    nw = nc * ns
    bw = b // nw
    mesh = plsc.VectorSubcoreMesh(core_axis_name="c", subcore_axis_name="s")

    @functools.partial(
        pl.kernel,
        mesh=mesh,
        out_type=jax.ShapeDtypeStruct((h, d, b), jnp.float32),
        compiler_params=pltpu.CompilerParams(
            use_tc_tiling_on_sc=False, needs_layout_passes=False
        ),
        scratch_types=[
            pltpu.VMEM((h, bw), jnp.int32),
            pltpu.VMEM((2, bw, d), jnp.float32),
            pltpu.VMEM((2, d, bw), jnp.float32),
        ]
        + [pltpu.SemaphoreType.DMA] * 4,
    )
    def k(xt_hbm, table_hbm, out_hbm, xv, rows, tst, g0, g1, s0, s1):
        gsem = (g0, g1)
        ssem = (s0, s1)
        wid = lax.axis_index("s") * nc + lax.axis_index("c")
        b0 = wid * bw
        pltpu.sync_copy(xt_hbm.at[:, pl.ds(b0, bw)], xv)

        def fire_gather(hh, buf):
            pltpu.async_copy(table_hbm.at[xv.at[hh]], rows.at[buf], gsem[buf])

        def wait_gather(hh, buf):
            pltpu.make_async_copy(
                table_hbm.at[xv.at[hh]], rows.at[buf], gsem[buf]
            ).wait()

        def fire_store(hh, buf):
            pltpu.async_copy(
                tst.at[buf], out_hbm.at[hh, :, pl.ds(b0, bw)], ssem[buf]
            )

        def wait_store(hh, buf):
            pltpu.make_async_copy(
                tst.at[buf], out_hbm.at[hh, :, pl.ds(b0, bw)], ssem[buf]
            ).wait()

        def transpose(buf):
            src = rows.at[buf]
            dst = tst.at[buf]

            def col(c, carry):
                for kk in range(bw // _L):
                    bl = lax.iota(jnp.int32, _L) + kk * _L
                    vals = plsc.load_gather(src, [bl, jnp.full((_L,), c, jnp.int32)])
                    dst[c, pl.ds(kk * _L, _L)] = vals
                return carry

            lax.fori_loop(0, d, col, 0)

        fire_gather(0, 0)
        fire_gather(1, 1)
        for u in range(2):
            wait_gather(u, u)
            transpose(u)
            fire_store(u, u)
            fire_gather(u + 2, u)

        def body(g, carry):
            for u in range(2):
                hh = 2 * g + u
                wait_gather(hh, u)
                wait_store(hh - 2, u)
                transpose(u)
                fire_store(hh, u)
                fire_gather(hh + 2, u)
            return carry

        lax.fori_loop(1, (h - 2) // 2, body, 0)
        for u in range(2):
            hh = h - 2 + u
            wait_gather(hh, u)
            wait_store(hh - 2, u)
            transpose(u)
            fire_store(hh, u)
        for u in range(2):
            wait_store(h - 2 + u, u)

    return k


def kernel(x, table):
    b, h = x.shape
    v, d = table.shape
    xt = jnp.transpose(x).astype(jnp.int32)
    out_t = _make_kernel(b, h, v, d)(xt, table)
    return jnp.transpose(out_t, (2, 0, 1))


# speedup vs baseline: 1.1163x; 1.1163x over previous
import functools

import jax
import jax.numpy as jnp
from jax import lax
from jax.experimental import pallas as pl
from jax.experimental.pallas import tpu as pltpu
from jax.experimental.pallas import tpu_sc as plsc

_L = 16


@functools.cache
def _make_kernel(b, h, v, d):
    info = plsc.get_sparse_core_info()
    nc, ns = info.num_cores, info.num_subcores
    nw = nc * ns
    bw = b // nw
    mesh = plsc.VectorSubcoreMesh(core_axis_name="c", subcore_axis_name="s")

    @functools.partial(
        pl.kernel,
        mesh=mesh,
        out_type=jax.ShapeDtypeStruct((h, d, b), jnp.float32),
        compiler_params=pltpu.CompilerParams(
            use_tc_tiling_on_sc=False, needs_layout_passes=False
        ),
        scratch_types=[
            pltpu.VMEM((h, bw), jnp.int32),
            pltpu.VMEM((2, bw, d), jnp.float32),
            pltpu.VMEM((2, d, bw), jnp.float32),
        ]
        + [pltpu.SemaphoreType.DMA] * 4,
    )
    def k(xt_hbm, table_hbm, out_hbm, xv, rows, tst, g0, g1, s0, s1):
        gsem = (g0, g1)
        ssem = (s0, s1)
        wid = lax.axis_index("s") * nc + lax.axis_index("c")
        b0 = wid * bw
        pltpu.sync_copy(xt_hbm.at[:, pl.ds(b0, bw)], xv)

        def fire_gather(hh, buf):
            pltpu.async_copy(table_hbm.at[xv.at[hh]], rows.at[buf], gsem[buf])

        def wait_gather(hh, buf):
            pltpu.make_async_copy(
                table_hbm.at[xv.at[hh]], rows.at[buf], gsem[buf]
            ).wait()

        def fire_store(hh, buf):
            pltpu.async_copy(
                tst.at[buf], out_hbm.at[hh, :, pl.ds(b0, bw)], ssem[buf]
            )

        def wait_store(hh, buf):
            pltpu.make_async_copy(
                tst.at[buf], out_hbm.at[hh, :, pl.ds(b0, bw)], ssem[buf]
            ).wait()

        cqs = [lax.iota(jnp.int32, _L) + _L * q for q in range(d // _L)]

        def transpose(buf):
            src = rows.at[buf]
            dst = tst.at[buf]
            for j in range(bw):
                jv = jnp.full((_L,), j, jnp.int32)
                for q in range(d // _L):
                    vals = src[j, pl.ds(_L * q, _L)]
                    plsc.store_scatter(dst, [cqs[q], jv], vals)

        fire_gather(0, 0)
        fire_gather(1, 1)
        for u in range(2):
            wait_gather(u, u)
            transpose(u)
            fire_store(u, u)
            fire_gather(u + 2, u)

        def body(g, carry):
            for u in range(2):
                hh = 2 * g + u
                wait_gather(hh, u)
                wait_store(hh - 2, u)
                transpose(u)
                fire_store(hh, u)
                fire_gather(hh + 2, u)
            return carry

        lax.fori_loop(1, (h - 2) // 2, body, 0)
        for u in range(2):
            hh = h - 2 + u
            wait_gather(hh, u)
            wait_store(hh - 2, u)
            transpose(u)
            fire_store(hh, u)
        for u in range(2):
            wait_store(h - 2 + u, u)

    return k


def kernel(x, table):
    b, h = x.shape
    v, d = table.shape
    xt = jnp.transpose(x).astype(jnp.int32)
    out_t = _make_kernel(b, h, v, d)(xt, table)
    return jnp.transpose(out_t, (2, 0, 1))


# speedup vs baseline: 1.1399x; 1.0211x over previous
import functools

import jax
import jax.numpy as jnp
from jax import lax
from jax.experimental import pallas as pl
from jax.experimental.pallas import tpu as pltpu
from jax.experimental.pallas import tpu_sc as plsc

_L = 16


@functools.cache
def _make_kernel(b, h, v, d):
    info = plsc.get_sparse_core_info()
    nc, ns = info.num_cores, info.num_subcores
    nw = nc * ns
    bw = b // nw
    mesh = plsc.VectorSubcoreMesh(core_axis_name="c", subcore_axis_name="s")

    @functools.partial(
        pl.kernel,
        mesh=mesh,
        out_type=jax.ShapeDtypeStruct((h, d, b), jnp.float32),
        compiler_params=pltpu.CompilerParams(
            use_tc_tiling_on_sc=False, needs_layout_passes=False
        ),
        scratch_types=[
            pltpu.VMEM((h, bw), jnp.int32),
            pltpu.VMEM((2, bw, d), jnp.float32),
            pltpu.VMEM((2, d, bw), jnp.float32),
        ]
        + [pltpu.SemaphoreType.DMA] * 4,
    )
    def k(xt_hbm, table_hbm, out_hbm, xv, rows, tst, g0, g1, s0, s1):
        gsem = (g0, g1)
        ssem = (s0, s1)
        wid = lax.axis_index("s") * nc + lax.axis_index("c")
        b0 = wid * bw
        pltpu.sync_copy(xt_hbm.at[:, pl.ds(b0, bw)], xv)

        def fire_gather(hh, buf):
            pltpu.async_copy(table_hbm.at[xv.at[hh]], rows.at[buf], gsem[buf])

        def wait_gather(hh, buf):
            pltpu.make_async_copy(
                table_hbm.at[xv.at[hh]], rows.at[buf], gsem[buf]
            ).wait()

        def fire_store(hh, buf):
            pltpu.async_copy(
                tst.at[buf], out_hbm.at[hh, :, pl.ds(b0, bw)], ssem[buf]
            )

        def wait_store(hh, buf):
            pltpu.make_async_copy(
                tst.at[buf], out_hbm.at[hh, :, pl.ds(b0, bw)], ssem[buf]
            ).wait()

        cqs = [lax.iota(jnp.int32, _L) + _L * q for q in range(d // _L)]

        def transpose(buf):
            src = rows.at[buf]
            dst = tst.at[buf]
            nq = d // _L
            for j in range(0, bw, 4):
                vals = [
                    [src[j + i, pl.ds(_L * q, _L)] for q in range(nq)]
                    for i in range(4)
                ]
                jvs = [jnp.full((_L,), j + i, jnp.int32) for i in range(4)]
                for i in range(4):
                    for q in range(nq):
                        plsc.store_scatter(dst, [cqs[q], jvs[i]], vals[i][q])

        fire_gather(0, 0)
        fire_gather(1, 1)
        for u in range(2):
            wait_gather(u, u)
            transpose(u)
            fire_store(u, u)
            fire_gather(u + 2, u)

        def body(g, carry):
            for u in range(2):
                hh = 2 * g + u
                wait_gather(hh, u)
                wait_store(hh - 2, u)
                transpose(u)
                fire_store(hh, u)
                fire_gather(hh + 2, u)
            return carry

        lax.fori_loop(1, (h - 2) // 2, body, 0)
        for u in range(2):
            hh = h - 2 + u
            wait_gather(hh, u)
            wait_store(hh - 2, u)
            transpose(u)
            fire_store(hh, u)
        for u in range(2):
            wait_store(h - 2 + u, u)

    return k


def kernel(x, table):
    b, h = x.shape
    v, d = table.shape
    xt = jnp.transpose(x).astype(jnp.int32)
    out_t = _make_kernel(b, h, v, d)(xt, table)
    return jnp.transpose(out_t, (2, 0, 1))


# speedup vs baseline: 1.1537x; 1.0121x over previous
import functools

import jax
import jax.numpy as jnp
from jax import lax
from jax.experimental import pallas as pl
from jax.experimental.pallas import tpu as pltpu
from jax.experimental.pallas import tpu_sc as plsc

_L = 16


@functools.cache
def _make_kernel(b, h, v, d):
    info = plsc.get_sparse_core_info()
    nc, ns = info.num_cores, info.num_subcores
    nw = nc * ns
    bw = b // nw
    mesh = plsc.VectorSubcoreMesh(core_axis_name="c", subcore_axis_name="s")

    @functools.partial(
        pl.kernel,
        mesh=mesh,
        out_type=jax.ShapeDtypeStruct((h, d, b), jnp.float32),
        compiler_params=pltpu.CompilerParams(
            use_tc_tiling_on_sc=False, needs_layout_passes=False
        ),
        scratch_types=[
            pltpu.VMEM((h, bw), jnp.int32),
            pltpu.VMEM((4, bw, d), jnp.float32),
            pltpu.VMEM((2, d, bw), jnp.float32),
        ]
        + [pltpu.SemaphoreType.DMA] * 6,
    )
    def k(xt_hbm, table_hbm, out_hbm, xv, rows, tst, g0, g1, g2, g3, s0, s1):
        gsem = (g0, g1, g2, g3)
        ssem = (s0, s1)
        wid = lax.axis_index("s") * nc + lax.axis_index("c")
        b0 = wid * bw
        pltpu.sync_copy(xt_hbm.at[:, pl.ds(b0, bw)], xv)

        def fire_gather(hh, buf):
            pltpu.async_copy(table_hbm.at[xv.at[hh]], rows.at[buf], gsem[buf])

        def wait_gather(hh, buf):
            pltpu.make_async_copy(
                table_hbm.at[xv.at[hh]], rows.at[buf], gsem[buf]
            ).wait()

        def fire_store(hh, buf):
            pltpu.async_copy(
                tst.at[buf], out_hbm.at[hh, :, pl.ds(b0, bw)], ssem[buf]
            )

        def wait_store(hh, buf):
            pltpu.make_async_copy(
                tst.at[buf], out_hbm.at[hh, :, pl.ds(b0, bw)], ssem[buf]
            ).wait()

        cqs = [lax.iota(jnp.int32, _L) + _L * q for q in range(d // _L)]

        def transpose(gbuf, tbuf):
            src = rows.at[gbuf]
            dst = tst.at[tbuf]
            nq = d // _L

            def grp(i, carry):
                j0 = i * 32
                for j in range(0, 32, 4):
                    vals = [
                        [src[j0 + j + u, pl.ds(_L * q, _L)] for q in range(nq)]
                        for u in range(4)
                    ]
                    jvs = [
                        jnp.full((_L,), j + u, jnp.int32) + j0 for u in range(4)
                    ]
                    for u in range(4):
                        for q in range(nq):
                            plsc.store_scatter(dst, [cqs[q], jvs[u]], vals[u][q])
                return carry

            lax.fori_loop(0, bw // 32, grp, 0)

        def step(hh, u, wait_st, fire_g):
            wait_gather(hh, u)
            if wait_st:
                wait_store(hh - 2, u % 2)
            transpose(u, u % 2)
            fire_store(hh, u % 2)
            if fire_g:
                fire_gather(hh + 3, (u + 3) % 4)

        for hh in range(3):
            fire_gather(hh, hh)
        for hh in range(4):
            step(hh, hh, wait_st=hh >= 2, fire_g=True)

        def body(gidx, carry):
            h0 = 4 + gidx * 4
            for u in range(4):
                step(h0 + u, u, wait_st=True, fire_g=True)
            return carry

        lax.fori_loop(0, (h - 8) // 4, body, 0)
        for hh in range(h - 4, h):
            step(hh, hh % 4, wait_st=True, fire_g=hh + 3 < h)
        for u in range(2):
            wait_store(h - 2 + u, u)

    return k


def kernel(x, table):
    b, h = x.shape
    v, d = table.shape
    xt = jnp.transpose(x).astype(jnp.int32)
    out_t = _make_kernel(b, h, v, d)(xt, table)
    return jnp.transpose(out_t, (2, 0, 1))


# speedup vs baseline: 1.7731x; 1.5369x over previous
import functools

import jax
import jax.numpy as jnp
from jax import lax
from jax.experimental import pallas as pl
from jax.experimental.pallas import tpu as pltpu
from jax.experimental.pallas import tpu_sc as plsc

_L = 16


@functools.cache
def _make_kernel(b, h, v, d):
    info = plsc.get_sparse_core_info()
    nc, ns = info.num_cores, info.num_subcores
    nw = nc * ns
    bw = b // nw
    mesh = plsc.VectorSubcoreMesh(core_axis_name="c", subcore_axis_name="s")

    @functools.partial(
        pl.kernel,
        mesh=mesh,
        out_type=jax.ShapeDtypeStruct((h, d, b), jnp.float32),
        compiler_params=pltpu.CompilerParams(
            use_tc_tiling_on_sc=False, needs_layout_passes=False
        ),
        scratch_types=[
            pltpu.VMEM((h, bw), jnp.int32),
            pltpu.VMEM((4, bw, d), jnp.float32),
            pltpu.VMEM((2, d, bw + 1), jnp.float32),
        ]
        + [pltpu.SemaphoreType.DMA] * 6,
    )
    def k(xt_hbm, table_hbm, out_hbm, xv, rows, tst, g0, g1, g2, g3, s0, s1):
        gsem = (g0, g1, g2, g3)
        ssem = (s0, s1)
        wid = lax.axis_index("s") * nc + lax.axis_index("c")
        b0 = wid * bw
        pltpu.sync_copy(xt_hbm.at[:, pl.ds(b0, bw)], xv)

        def fire_gather(hh, buf):
            pltpu.async_copy(table_hbm.at[xv.at[hh]], rows.at[buf], gsem[buf])

        def wait_gather(hh, buf):
            pltpu.make_async_copy(
                table_hbm.at[xv.at[hh]], rows.at[buf], gsem[buf]
            ).wait()

        def fire_store(hh, buf):
            pltpu.async_copy(
                tst.at[buf, :, pl.ds(0, bw)],
                out_hbm.at[hh, :, pl.ds(b0, bw)],
                ssem[buf],
            )

        def wait_store(hh, buf):
            pltpu.make_async_copy(
                tst.at[buf, :, pl.ds(0, bw)],
                out_hbm.at[hh, :, pl.ds(b0, bw)],
                ssem[buf],
            ).wait()

        cqs = [lax.iota(jnp.int32, _L) + _L * q for q in range(d // _L)]

        def transpose(gbuf, tbuf):
            src = rows.at[gbuf]
            dst = tst.at[tbuf]
            nq = d // _L

            def grp(i, carry):
                j0 = i * 32
                for j in range(0, 32, 4):
                    vals = [
                        [src[j0 + j + u, pl.ds(_L * q, _L)] for q in range(nq)]
                        for u in range(4)
                    ]
                    jvs = [
                        jnp.full((_L,), j + u, jnp.int32) + j0 for u in range(4)
                    ]
                    for u in range(4):
                        for q in range(nq):
                            plsc.store_scatter(dst, [cqs[q], jvs[u]], vals[u][q])
                return carry

            lax.fori_loop(0, bw // 32, grp, 0)

        def step(hh, u, wait_st, fire_g):
            wait_gather(hh, u)
            if wait_st:
                wait_store(hh - 2, u % 2)
            transpose(u, u % 2)
            fire_store(hh, u % 2)
            if fire_g:
                fire_gather(hh + 3, (u + 3) % 4)

        for hh in range(3):
            fire_gather(hh, hh)
        for hh in range(4):
            step(hh, hh, wait_st=hh >= 2, fire_g=True)

        def body(gidx, carry):
            h0 = 4 + gidx * 4
            for u in range(4):
                step(h0 + u, u, wait_st=True, fire_g=True)
            return carry

        lax.fori_loop(0, (h - 8) // 4, body, 0)
        for hh in range(h - 4, h):
            step(hh, hh % 4, wait_st=True, fire_g=hh + 3 < h)
        for u in range(2):
            wait_store(h - 2 + u, u)

    return k


def kernel(x, table):
    b, h = x.shape
    v, d = table.shape
    xt = jnp.transpose(x).astype(jnp.int32)
    out_t = _make_kernel(b, h, v, d)(xt, table)
    return jnp.transpose(out_t, (2, 0, 1))
